# baseline (speedup 1.0000x reference)
import functools

import jax
import jax.numpy as jnp
from jax import lax
from jax.experimental import pallas as pl
from jax.experimental.pallas import tpu as pltpu
from jax.experimental.pallas import tpu_sc as plsc

VOCAB = 1000000
DIM = 64
B = 4096
L = 50

NC = 2
NS = 16
LANES = 16
NW = NC * NS

ROWS = B * L
RPW = ROWS // NW
SLAB = 128
GR = 128
NCHUNK = RPW // GR
NGRP = GR // LANES
LPC = GR // SLAB

TBLK = 4096
NTBLK = (VOCAB + TBLK - 1) // TBLK
PREC = TBLK // 2
PTAB = NTBLK * PREC

_MAGIC = 0x5F3759DF


def _pack_body(tt_ref, out_ref):
    xt = tt_ref[...].T
    out_ref[...] = jnp.concatenate([xt[:PREC], xt[PREC:]], axis=1)


_pack_table = pl.pallas_call(
    _pack_body,
    grid=(NTBLK,),
    in_specs=[pl.BlockSpec((DIM, TBLK), lambda c: (0, c))],
    out_specs=pl.BlockSpec((PREC, DIM * 2), lambda c: (c, 0)),
    out_shape=jax.ShapeDtypeStruct((PTAB, DIM * 2), jnp.float32),
)


def _rsqrt(ssv):
    bits = plsc.bitcast(ssv, jnp.int32)
    y = plsc.bitcast(_MAGIC - (bits >> 1), jnp.float32)
    for _ in range(3):
        y = y * (1.5 - 0.5 * (ssv * y) * y)
    return y


_mesh = plsc.VectorSubcoreMesh(core_axis_name="c", subcore_axis_name="s")


@functools.partial(
    pl.kernel,
    mesh=_mesh,
    out_type=jax.ShapeDtypeStruct((L, DIM, B), jnp.float32),
    scratch_types=[
        pltpu.VMEM((RPW,), jnp.int32),
        pltpu.VMEM((RPW,), jnp.int32),
        pltpu.VMEM((GR, DIM * 2), jnp.float32),
        pltpu.VMEM((GR, DIM * 2), jnp.float32),
        pltpu.VMEM((GR, DIM * 2), jnp.float32),
        pltpu.VMEM((LPC * DIM, SLAB), jnp.float32),
        pltpu.SemaphoreType.DMA,
        pltpu.SemaphoreType.DMA,
        pltpu.SemaphoreType.DMA,
        pltpu.SemaphoreType.DMA,
    ],
    compiler_params=pltpu.CompilerParams(needs_layout_passes=False),
)
def _sc_lookup(idx_hbm, sel_hbm, table_hbm, out_hbm,
               idx_v, sel_v, buf0, buf1, buf2, obuf,
               gsem0, gsem1, gsem2, osem):
    wid = lax.axis_index("s") * NC + lax.axis_index("c")
    col0 = wid * SLAB

    pltpu.sync_copy(idx_hbm.at[wid], idx_v)
    pltpu.sync_copy(sel_hbm.at[wid], sel_v)

    def compute(j, buf):
        @plsc.parallel_loop(0, NGRP, 1, unroll=2)
        def grp_body(g):
            iota = lax.iota(jnp.int32, LANES)
            rows = g * LANES + iota
            sel = sel_v[pl.ds(j * GR + g * LANES, LANES)]
            hcol = (sel & 1) * DIM
            lane0 = g * LANES
            acc = [None] * 4
            for k in range(DIM):
                v = plsc.load_gather(buf, [rows, hcol + k])
                obuf[k, pl.ds(lane0, LANES)] = v
                a = acc[k & 3]
                acc[k & 3] = v * v if a is None else a + v * v
            ss = (acc[0] + acc[1]) + (acc[2] + acc[3])
            y = _rsqrt(ss)
            norm = ss * y
            bm = jnp.where(sel >= 2, 0.0, 1.0)
            inv = jnp.where(norm >= 1e-6, y, 1e6) * bm
            for k in range(DIM):
                w = obuf[k, pl.ds(lane0, LANES)]
                obuf[k, pl.ds(lane0, LANES)] = w * inv

    def gather(j, buf, sem):
        return pltpu.async_copy(
            table_hbm.at[idx_v.at[pl.ds(j * GR, GR)]], buf, sem)

    def writeback(j):
        for p in range(LPC):
            pltpu.async_copy(
                obuf.at[pl.ds(p * DIM, DIM)],
                out_hbm.at[LPC * j + p, :, pl.ds(col0, SLAB)], osem,
            )

    def wait_writeback(j):
        for p in range(LPC):
            pltpu.make_async_copy(
                obuf.at[pl.ds(p * DIM, DIM)],
                out_hbm.at[LPC * j + p, :, pl.ds(col0, SLAB)], osem,
            ).wait()

    gather(0, buf0, gsem0)
    gather(1, buf1, gsem1)
    gather(2, buf2, gsem2)

    def stage(do_wait, j, buf, gsem):
        pltpu.make_async_copy(
            table_hbm.at[idx_v.at[pl.ds(j * GR, GR)]], buf, gsem).wait()

        @pl.when(do_wait)
        def _():
            wait_writeback(j)

        compute(j, buf)
        writeback(j)
        gather(jnp.minimum(j + 3, NCHUNK - 1), buf, gsem)

    def outer(t, carry):
        stage(t > 0, jnp.minimum(3 * t, NCHUNK - 1), buf0, gsem0)
        stage(t >= 0, jnp.minimum(3 * t + 1, NCHUNK - 1), buf1, gsem1)
        stage(t >= 0, jnp.minimum(3 * t + 2, NCHUNK - 1), buf2, gsem2)
        return carry

    lax.fori_loop(0, (NCHUNK + 2) // 3, outer, 0)

    pltpu.make_async_copy(
        table_hbm.at[idx_v.at[pl.ds(0, GR)]], buf0, gsem0).wait()
    pltpu.make_async_copy(
        table_hbm.at[idx_v.at[pl.ds(0, GR)]], buf1, gsem1).wait()
    pltpu.make_async_copy(
        table_hbm.at[idx_v.at[pl.ds(0, GR)]], buf2, gsem2).wait()
    wait_writeback(0)


def kernel(indices, mask, table):
    slab = indices.reshape(NW, SLAB, L).transpose(0, 2, 1)
    blk = slab >> 12
    q = slab & (TBLK - 1)
    idx3 = ((blk << 11) | (q & (PREC - 1))).reshape(NW, RPW)
    mslab = (mask.astype(jnp.int32).reshape(NW, SLAB, L)
             .transpose(0, 2, 1).reshape(NW, SLAB * L))
    sel3 = ((q >> 11).reshape(NW, RPW)) | (mslab << 1)
    table2 = _pack_table(table.T)
    out = _sc_lookup(idx3, sel3, table2)
    return out.transpose(2, 0, 1)


# speedup vs baseline: 1.1136x; 1.1136x over previous
import functools

import jax
import jax.numpy as jnp
from jax import lax
from jax.experimental import pallas as pl
from jax.experimental.pallas import tpu as pltpu
from jax.experimental.pallas import tpu_sc as plsc

VOCAB = 1000000
DIM = 64
B = 4096
L = 50

NC = 2
NS = 16
LANES = 16
NW = NC * NS

ROWS = B * L
RPW = ROWS // NW
SLAB = 128
GR = 128
NCHUNK = RPW // GR
NGRP = GR // LANES
LPC = GR // SLAB

TBLK = 8192
NTBLK = (VOCAB + TBLK - 1) // TBLK
PREC = TBLK // 2
PTAB = NTBLK * PREC
TB_SH = TBLK.bit_length() - 1
PR_SH = PREC.bit_length() - 1

_MAGIC = 0x5F3759DF


def _pack_body(tt_ref, out_ref):
    xt = tt_ref[...].T
    out_ref[...] = jnp.concatenate([xt[:PREC], xt[PREC:]], axis=1)


_pack_table = pl.pallas_call(
    _pack_body,
    grid=(NTBLK,),
    in_specs=[pl.BlockSpec((DIM, TBLK), lambda c: (0, c))],
    out_specs=pl.BlockSpec((PREC, DIM * 2), lambda c: (c, 0)),
    out_shape=jax.ShapeDtypeStruct((PTAB, DIM * 2), jnp.float32),
)


def _rsqrt(ssv):
    bits = plsc.bitcast(ssv, jnp.int32)
    y = plsc.bitcast(_MAGIC - (bits >> 1), jnp.float32)
    for _ in range(3):
        y = y * (1.5 - 0.5 * (ssv * y) * y)
    return y


_mesh = plsc.VectorSubcoreMesh(core_axis_name="c", subcore_axis_name="s")


@functools.partial(
    pl.kernel,
    mesh=_mesh,
    out_type=jax.ShapeDtypeStruct((L, DIM, B), jnp.float32),
    scratch_types=[
        pltpu.VMEM((RPW,), jnp.int32),
        pltpu.VMEM((RPW,), jnp.int32),
        pltpu.VMEM((GR, DIM * 2), jnp.float32),
        pltpu.VMEM((GR, DIM * 2), jnp.float32),
        pltpu.VMEM((LPC * DIM, SLAB), jnp.float32),
        pltpu.SemaphoreType.DMA,
        pltpu.SemaphoreType.DMA,
        pltpu.SemaphoreType.DMA,
    ],
    compiler_params=pltpu.CompilerParams(needs_layout_passes=False),
)
def _sc_lookup(idx_hbm, sel_hbm, table_hbm, out_hbm,
               idx_v, sel_v, buf0, buf1, obuf,
               gsem0, gsem1, osem):
    wid = lax.axis_index("s") * NC + lax.axis_index("c")
    col0 = wid * SLAB

    pltpu.sync_copy(idx_hbm.at[wid], idx_v)
    pltpu.sync_copy(sel_hbm.at[wid], sel_v)

    def compute(j, buf):
        @plsc.parallel_loop(0, NGRP, 1, unroll=2)
        def grp_body(g):
            iota = lax.iota(jnp.int32, LANES)
            rows = g * LANES + iota
            sel = sel_v[pl.ds(j * GR + g * LANES, LANES)]
            hcol = (sel & 1) * DIM
            lane0 = g * LANES
            acc = [None] * 4
            for k in range(DIM):
                v = plsc.load_gather(buf, [rows, hcol + k])
                obuf[k, pl.ds(lane0, LANES)] = v
                a = acc[k & 3]
                acc[k & 3] = v * v if a is None else a + v * v
            ss = (acc[0] + acc[1]) + (acc[2] + acc[3])
            y = _rsqrt(ss)
            norm = ss * y
            bm = jnp.where(sel >= 2, 0.0, 1.0)
            inv = jnp.where(norm >= 1e-6, y, 1e6) * bm
            for k in range(DIM):
                w = obuf[k, pl.ds(lane0, LANES)]
                obuf[k, pl.ds(lane0, LANES)] = w * inv

    def gather(j, buf, sem):
        return pltpu.async_copy(
            table_hbm.at[idx_v.at[pl.ds(j * GR, GR)]], buf, sem)

    def writeback(j):
        for p in range(LPC):
            pltpu.async_copy(
                obuf.at[pl.ds(p * DIM, DIM)],
                out_hbm.at[LPC * j + p, :, pl.ds(col0, SLAB)], osem,
            )

    def wait_writeback(j):
        for p in range(LPC):
            pltpu.make_async_copy(
                obuf.at[pl.ds(p * DIM, DIM)],
                out_hbm.at[LPC * j + p, :, pl.ds(col0, SLAB)], osem,
            ).wait()

    gather(0, buf0, gsem0)
    gather(1, buf1, gsem1)

    def stage(do_wait, j, buf, gsem):
        pltpu.make_async_copy(
            table_hbm.at[idx_v.at[pl.ds(j * GR, GR)]], buf, gsem).wait()

        @pl.when(do_wait)
        def _():
            wait_writeback(j)

        compute(j, buf)
        writeback(j)
        gather(jnp.minimum(j + 2, NCHUNK - 1), buf, gsem)

    def outer(t, carry):
        stage(t > 0, 2 * t, buf0, gsem0)
        stage(t >= 0, 2 * t + 1, buf1, gsem1)
        return carry

    lax.fori_loop(0, NCHUNK // 2, outer, 0)
    tail = jnp.int32(NCHUNK // 2)
    stage(tail > 0, NCHUNK - 1, buf0, gsem0)

    pltpu.make_async_copy(
        table_hbm.at[idx_v.at[pl.ds(0, GR)]], buf0, gsem0).wait()
    pltpu.make_async_copy(
        table_hbm.at[idx_v.at[pl.ds(0, GR)]], buf1, gsem1).wait()
    wait_writeback(0)


def kernel(indices, mask, table):
    slab = indices.reshape(NW, SLAB, L).transpose(0, 2, 1)
    blk = slab >> TB_SH
    q = slab & (TBLK - 1)
    idx3 = ((blk << PR_SH) | (q & (PREC - 1))).reshape(NW, RPW)
    mslab = (mask.astype(jnp.int32).reshape(NW, SLAB, L)
             .transpose(0, 2, 1).reshape(NW, SLAB * L))
    sel3 = ((q >> PR_SH).reshape(NW, RPW)) | (mslab << 1)
    table2 = _pack_table(table.T)
    out = _sc_lookup(idx3, sel3, table2)
    return out.transpose(2, 0, 1)


# speedup vs baseline: 1.1707x; 1.0512x over previous
import functools

import jax
import jax.numpy as jnp
from jax import lax
from jax.experimental import pallas as pl
from jax.experimental.pallas import tpu as pltpu
from jax.experimental.pallas import tpu_sc as plsc

VOCAB = 1000000
DIM = 64
B = 4096
L = 50

NC = 2
NS = 16
LANES = 16
NW = NC * NS

ROWS = B * L
RPW = ROWS // NW
SLAB = 128
GR = 128
NCHUNK = RPW // GR
NGRP = GR // LANES
LPC = GR // SLAB

TBLK = 16384
NTBLK = (VOCAB + TBLK - 1) // TBLK
PREC = TBLK // 2
PTAB = NTBLK * PREC
TB_SH = TBLK.bit_length() - 1
PR_SH = PREC.bit_length() - 1

_MAGIC = 0x5F3759DF


def _pack_body(tt_ref, out_ref):
    xt = tt_ref[...].T
    out_ref[...] = jnp.concatenate([xt[:PREC], xt[PREC:]], axis=1)


_pack_table = pl.pallas_call(
    _pack_body,
    grid=(NTBLK,),
    in_specs=[pl.BlockSpec((DIM, TBLK), lambda c: (0, c))],
    out_specs=pl.BlockSpec((PREC, DIM * 2), lambda c: (c, 0)),
    out_shape=jax.ShapeDtypeStruct((PTAB, DIM * 2), jnp.float32),
)


def _rsqrt(ssv):
    bits = plsc.bitcast(ssv, jnp.int32)
    y = plsc.bitcast(_MAGIC - (bits >> 1), jnp.float32)
    for _ in range(3):
        y = y * (1.5 - 0.5 * (ssv * y) * y)
    return y


_mesh = plsc.VectorSubcoreMesh(core_axis_name="c", subcore_axis_name="s")


@functools.partial(
    pl.kernel,
    mesh=_mesh,
    out_type=jax.ShapeDtypeStruct((L, DIM, B), jnp.float32),
    scratch_types=[
        pltpu.VMEM((RPW,), jnp.int32),
        pltpu.VMEM((RPW,), jnp.int32),
        pltpu.VMEM((GR, DIM * 2), jnp.float32),
        pltpu.VMEM((GR, DIM * 2), jnp.float32),
        pltpu.VMEM((LPC * DIM, SLAB), jnp.float32),
        pltpu.SemaphoreType.DMA,
        pltpu.SemaphoreType.DMA,
        pltpu.SemaphoreType.DMA,
    ],
    compiler_params=pltpu.CompilerParams(needs_layout_passes=False),
)
def _sc_lookup(idx_hbm, sel_hbm, table_hbm, out_hbm,
               idx_v, sel_v, buf0, buf1, obuf,
               gsem0, gsem1, osem):
    wid = lax.axis_index("s") * NC + lax.axis_index("c")
    col0 = wid * SLAB

    pltpu.sync_copy(idx_hbm.at[wid], idx_v)
    pltpu.sync_copy(sel_hbm.at[wid], sel_v)

    def compute(j, buf):
        @plsc.parallel_loop(0, NGRP, 1, unroll=2)
        def grp_body(g):
            iota = lax.iota(jnp.int32, LANES)
            rows = g * LANES + iota
            sel = sel_v[pl.ds(j * GR + g * LANES, LANES)]
            hcol = (sel & 1) * DIM
            lane0 = g * LANES
            acc = [None] * 4
            for k in range(DIM):
                v = plsc.load_gather(buf, [rows, hcol + k])
                obuf[k, pl.ds(lane0, LANES)] = v
                a = acc[k & 3]
                acc[k & 3] = v * v if a is None else a + v * v
            ss = (acc[0] + acc[1]) + (acc[2] + acc[3])
            y = _rsqrt(ss)
            norm = ss * y
            bm = jnp.where(sel >= 2, 0.0, 1.0)
            inv = jnp.where(norm >= 1e-6, y, 1e6) * bm
            for k in range(DIM):
                w = obuf[k, pl.ds(lane0, LANES)]
                obuf[k, pl.ds(lane0, LANES)] = w * inv

    def gather(j, buf, sem):
        return pltpu.async_copy(
            table_hbm.at[idx_v.at[pl.ds(j * GR, GR)]], buf, sem)

    def writeback(j):
        for p in range(LPC):
            pltpu.async_copy(
                obuf.at[pl.ds(p * DIM, DIM)],
                out_hbm.at[LPC * j + p, :, pl.ds(col0, SLAB)], osem,
            )

    def wait_writeback(j):
        for p in range(LPC):
            pltpu.make_async_copy(
                obuf.at[pl.ds(p * DIM, DIM)],
                out_hbm.at[LPC * j + p, :, pl.ds(col0, SLAB)], osem,
            ).wait()

    gather(0, buf0, gsem0)
    gather(1, buf1, gsem1)

    def stage(do_wait, j, buf, gsem):
        pltpu.make_async_copy(
            table_hbm.at[idx_v.at[pl.ds(j * GR, GR)]], buf, gsem).wait()

        @pl.when(do_wait)
        def _():
            wait_writeback(j)

        compute(j, buf)
        writeback(j)
        gather(jnp.minimum(j + 2, NCHUNK - 1), buf, gsem)

    def outer(t, carry):
        stage(t > 0, 2 * t, buf0, gsem0)
        stage(t >= 0, 2 * t + 1, buf1, gsem1)
        return carry

    lax.fori_loop(0, NCHUNK // 2, outer, 0)
    tail = jnp.int32(NCHUNK // 2)
    stage(tail > 0, NCHUNK - 1, buf0, gsem0)

    pltpu.make_async_copy(
        table_hbm.at[idx_v.at[pl.ds(0, GR)]], buf0, gsem0).wait()
    pltpu.make_async_copy(
        table_hbm.at[idx_v.at[pl.ds(0, GR)]], buf1, gsem1).wait()
    wait_writeback(0)


def kernel(indices, mask, table):
    slab = indices.reshape(NW, SLAB, L).transpose(0, 2, 1)
    blk = slab >> TB_SH
    q = slab & (TBLK - 1)
    idx3 = ((blk << PR_SH) | (q & (PREC - 1))).reshape(NW, RPW)
    mslab = (mask.astype(jnp.int32).reshape(NW, SLAB, L)
             .transpose(0, 2, 1).reshape(NW, SLAB * L))
    sel3 = ((q >> PR_SH).reshape(NW, RPW)) | (mslab << 1)
    table2 = _pack_table(table.T)
    out = _sc_lookup(idx3, sel3, table2)
    return out.transpose(2, 0, 1)


# speedup vs baseline: 1.1992x; 1.0243x over previous
import functools

import jax
import jax.numpy as jnp
from jax import lax
from jax.experimental import pallas as pl
from jax.experimental.pallas import tpu as pltpu
from jax.experimental.pallas import tpu_sc as plsc

VOCAB = 1000000
DIM = 64
B = 4096
L = 50

NC = 2
NS = 16
LANES = 16
NW = NC * NS

ROWS = B * L
RPW = ROWS // NW
SLAB = 128
GR = 128
NCHUNK = RPW // GR
NGRP = GR // LANES
LPC = GR // SLAB

TBLK = 32768
NTBLK = (VOCAB + TBLK - 1) // TBLK
PREC = TBLK // 2
PTAB = NTBLK * PREC
TB_SH = TBLK.bit_length() - 1
PR_SH = PREC.bit_length() - 1

_MAGIC = 0x5F3759DF


def _pack_body(tt_ref, out_ref):
    xt = tt_ref[...].T
    out_ref[...] = jnp.concatenate([xt[:PREC], xt[PREC:]], axis=1)


_pack_table = pl.pallas_call(
    _pack_body,
    grid=(NTBLK,),
    in_specs=[pl.BlockSpec((DIM, TBLK), lambda c: (0, c))],
    out_specs=pl.BlockSpec((PREC, DIM * 2), lambda c: (c, 0)),
    out_shape=jax.ShapeDtypeStruct((PTAB, DIM * 2), jnp.float32),
)


def _rsqrt(ssv):
    bits = plsc.bitcast(ssv, jnp.int32)
    y = plsc.bitcast(_MAGIC - (bits >> 1), jnp.float32)
    for _ in range(3):
        y = y * (1.5 - 0.5 * (ssv * y) * y)
    return y


_mesh = plsc.VectorSubcoreMesh(core_axis_name="c", subcore_axis_name="s")


@functools.partial(
    pl.kernel,
    mesh=_mesh,
    out_type=jax.ShapeDtypeStruct((L, DIM, B), jnp.float32),
    scratch_types=[
        pltpu.VMEM((RPW,), jnp.int32),
        pltpu.VMEM((RPW,), jnp.int32),
        pltpu.VMEM((GR, DIM * 2), jnp.float32),
        pltpu.VMEM((GR, DIM * 2), jnp.float32),
        pltpu.VMEM((LPC * DIM, SLAB), jnp.float32),
        pltpu.SemaphoreType.DMA,
        pltpu.SemaphoreType.DMA,
        pltpu.SemaphoreType.DMA,
    ],
    compiler_params=pltpu.CompilerParams(needs_layout_passes=False),
)
def _sc_lookup(idx_hbm, sel_hbm, table_hbm, out_hbm,
               idx_v, sel_v, buf0, buf1, obuf,
               gsem0, gsem1, osem):
    wid = lax.axis_index("s") * NC + lax.axis_index("c")
    col0 = wid * SLAB

    pltpu.sync_copy(idx_hbm.at[wid], idx_v)
    pltpu.sync_copy(sel_hbm.at[wid], sel_v)

    def compute(j, buf):
        @plsc.parallel_loop(0, NGRP, 1, unroll=2)
        def grp_body(g):
            iota = lax.iota(jnp.int32, LANES)
            rows = g * LANES + iota
            sel = sel_v[pl.ds(j * GR + g * LANES, LANES)]
            hcol = (sel & 1) * DIM
            lane0 = g * LANES
            acc = [None] * 4
            for k in range(DIM):
                v = plsc.load_gather(buf, [rows, hcol + k])
                obuf[k, pl.ds(lane0, LANES)] = v
                a = acc[k & 3]
                acc[k & 3] = v * v if a is None else a + v * v
            ss = (acc[0] + acc[1]) + (acc[2] + acc[3])
            y = _rsqrt(ss)
            norm = ss * y
            bm = jnp.where(sel >= 2, 0.0, 1.0)
            inv = jnp.where(norm >= 1e-6, y, 1e6) * bm
            for k in range(DIM):
                w = obuf[k, pl.ds(lane0, LANES)]
                obuf[k, pl.ds(lane0, LANES)] = w * inv

    def gather(j, buf, sem):
        return pltpu.async_copy(
            table_hbm.at[idx_v.at[pl.ds(j * GR, GR)]], buf, sem)

    def writeback(j):
        for p in range(LPC):
            pltpu.async_copy(
                obuf.at[pl.ds(p * DIM, DIM)],
                out_hbm.at[LPC * j + p, :, pl.ds(col0, SLAB)], osem,
            )

    def wait_writeback(j):
        for p in range(LPC):
            pltpu.make_async_copy(
                obuf.at[pl.ds(p * DIM, DIM)],
                out_hbm.at[LPC * j + p, :, pl.ds(col0, SLAB)], osem,
            ).wait()

    gather(0, buf0, gsem0)
    gather(1, buf1, gsem1)

    def stage(do_wait, j, buf, gsem):
        pltpu.make_async_copy(
            table_hbm.at[idx_v.at[pl.ds(j * GR, GR)]], buf, gsem).wait()

        @pl.when(do_wait)
        def _():
            wait_writeback(j)

        compute(j, buf)
        writeback(j)
        gather(jnp.minimum(j + 2, NCHUNK - 1), buf, gsem)

    def outer(t, carry):
        stage(t > 0, 2 * t, buf0, gsem0)
        stage(t >= 0, 2 * t + 1, buf1, gsem1)
        return carry

    lax.fori_loop(0, NCHUNK // 2, outer, 0)
    tail = jnp.int32(NCHUNK // 2)
    stage(tail > 0, NCHUNK - 1, buf0, gsem0)

    pltpu.make_async_copy(
        table_hbm.at[idx_v.at[pl.ds(0, GR)]], buf0, gsem0).wait()
    pltpu.make_async_copy(
        table_hbm.at[idx_v.at[pl.ds(0, GR)]], buf1, gsem1).wait()
    wait_writeback(0)


def kernel(indices, mask, table):
    slab = indices.reshape(NW, SLAB, L).transpose(0, 2, 1)
    blk = slab >> TB_SH
    q = slab & (TBLK - 1)
    idx3 = ((blk << PR_SH) | (q & (PREC - 1))).reshape(NW, RPW)
    mslab = (mask.astype(jnp.int32).reshape(NW, SLAB, L)
             .transpose(0, 2, 1).reshape(NW, SLAB * L))
    sel3 = ((q >> PR_SH).reshape(NW, RPW)) | (mslab << 1)
    table2 = _pack_table(table.T)
    out = _sc_lookup(idx3, sel3, table2)
    return out.transpose(2, 0, 1)
